# baseline (speedup 1.0000x reference)
import functools

import jax
import jax.numpy as jnp
from jax import lax
from jax.experimental import pallas as pl
from jax.experimental.pallas import tpu as pltpu
from jax.experimental.pallas import tpu_sc as plsc

NUM_HEADS = 16
VOCAB = 512
B_TOTAL = 8 * 512 * 512
NW = 32
CHUNK = 512
N_CHUNKS = B_TOTAL // CHUNK
CPW = N_CHUNKS // NW
NBUF = 7

_mesh = plsc.VectorSubcoreMesh(core_axis_name="c", subcore_axis_name="s")


@functools.partial(
    pl.kernel,
    mesh=_mesh,
    out_type=jax.ShapeDtypeStruct((8, 512, 2, 4, 8, 128), jnp.float32),
    scratch_types=[
        pltpu.VMEM((NBUF, CHUNK), jnp.int32),
        pltpu.VMEM((NBUF, CHUNK, NUM_HEADS), jnp.float32),
        pltpu.VMEM((NBUF, 64, 129), jnp.float32),
        pltpu.VMEM_SHARED((VOCAB, NUM_HEADS), jnp.float32),
        pltpu.SemaphoreType.DMA((NBUF,)),
        pltpu.SemaphoreType.DMA((NBUF,)),
        pltpu.SemaphoreType.DMA((NBUF,)),
    ],
    compiler_params=pltpu.CompilerParams(use_tc_tiling_on_sc=False,
                                         needs_layout_passes=False),
)
def _gather_kernel(table_hbm, idx_hbm, out_hbm, idx_v, rows_v, out_pad,
                   table_sh, idx_sem, gat_sem, wb_sem):
    sid = lax.axis_index("s")
    w = sid * 2 + lax.axis_index("c")
    c0 = w * CPW

    @pl.when(sid == 0)
    def _stage_table():
        pltpu.sync_copy(table_hbm, table_sh)

    plsc.subcore_barrier()

    iota16 = lax.broadcasted_iota(jnp.int32, (16,), 0)
    row_ids = [jnp.full((16,), ii * 16, jnp.int32) + iota16 for ii in range(4)]

    def decode(c):
        return c // 512, (c % 512) // 8, (c % 8) // 2, c % 2

    def start_idx(c, buf):
        pltpu.async_copy(idx_hbm.at[pl.ds(c * CHUNK, CHUNK)],
                         idx_v.at[buf], idx_sem.at[buf])

    def wait_idx(buf):
        pltpu.make_async_copy(idx_hbm.at[pl.ds(0, CHUNK)],
                              idx_v.at[buf], idx_sem.at[buf]).wait()

    def start_gathers(buf):
        pltpu.async_copy(table_sh.at[idx_v.at[buf]], rows_v.at[buf],
                         gat_sem.at[buf])

    def wait_gathers(buf):
        pltpu.make_async_copy(table_sh.at[idx_v.at[buf]], rows_v.at[buf],
                              gat_sem.at[buf]).wait()

    def start_wb(c, buf):
        b, it, jt, half = decode(c)
        for ii in range(4):
            for ht in range(2):
                pltpu.async_copy(
                    out_pad.at[buf, pl.ds(ii * 16 + ht * 8, 8),
                               pl.ds(0, 128)],
                    out_hbm.at[b, it * 8 + half * 4 + ii, ht, jt],
                    wb_sem.at[buf])

    def wait_wb(buf):
        for ii in range(4):
            for ht in range(2):
                pltpu.make_async_copy(
                    out_pad.at[buf, pl.ds(ii * 16 + ht * 8, 8),
                               pl.ds(0, 128)],
                    out_hbm.at[0, 0, ht, 0],
                    wb_sem.at[buf]).wait()

    def compute(buf):
        @plsc.parallel_loop(0, 128, 1, unroll=4)
        def _body(jj):
            col = jnp.full((16,), jj, jnp.int32)
            for ii in range(4):
                vals = rows_v[buf, ii * 128 + jj]
                plsc.store_scatter(out_pad.at[buf], [row_ids[ii], col], vals)

    def run_chunk(c, buf, skip_wb_wait, has1, hasn):
        if has1:
            wait_idx((buf + 1) % NBUF)
            start_gathers((buf + 1) % NBUF)
        wait_gathers(buf)
        if not skip_wb_wait:
            wait_wb(buf)
        compute(buf)
        start_wb(c, buf)
        if hasn:
            start_idx(c + NBUF, buf)

    for k in range(NBUF):
        start_idx(c0 + k, k)
    wait_idx(0)
    start_gathers(0)
    for k in range(NBUF):
        run_chunk(c0 + k, k, True, True, True)

    n_steady = (CPW - 2 * NBUF) // NBUF

    def rounds(r, carry):
        g = c0 + NBUF + NBUF * r
        for k in range(NBUF):
            run_chunk(g + k, k, False, True, True)
        return carry

    lax.fori_loop(0, n_steady, rounds, 0)

    for g in range(NBUF + n_steady * NBUF, CPW):
        run_chunk(c0 + g, g % NBUF, False, g + 1 < CPW, g + NBUF < CPW)
    for k in range(NBUF):
        wait_wb(k)


def kernel(dist, embedding_table):
    idx = (dist.astype(jnp.int32)
           .reshape(8, 64, 8, 4, 128)
           .transpose(0, 1, 3, 2, 4)
           .reshape(B_TOTAL))
    out = _gather_kernel(embedding_table, idx)
    return out.transpose(0, 1, 3, 5, 2, 4).reshape(8, 512, 512, NUM_HEADS)


# speedup vs baseline: 1.0333x; 1.0333x over previous
import functools

import jax
import jax.numpy as jnp
from jax import lax
from jax.experimental import pallas as pl
from jax.experimental.pallas import tpu as pltpu
from jax.experimental.pallas import tpu_sc as plsc

NUM_HEADS = 16
VOCAB = 512
B_TOTAL = 8 * 512 * 512
NW = 32
CHUNK = 512
N_CHUNKS = B_TOTAL // CHUNK
CPW = N_CHUNKS // NW
NBUF = 5

_mesh = plsc.VectorSubcoreMesh(core_axis_name="c", subcore_axis_name="s")


@functools.partial(
    pl.kernel,
    mesh=_mesh,
    out_type=jax.ShapeDtypeStruct((8, 512, 2, 4, 8, 128), jnp.float32),
    scratch_types=[
        pltpu.VMEM((NBUF, CHUNK), jnp.int32),
        pltpu.VMEM((NBUF, CHUNK, NUM_HEADS), jnp.float32),
        pltpu.VMEM((NBUF, 64, 129), jnp.float32),
        pltpu.VMEM_SHARED((VOCAB, NUM_HEADS), jnp.float32),
        pltpu.SemaphoreType.DMA((NBUF,)),
        pltpu.SemaphoreType.DMA((NBUF,)),
        pltpu.SemaphoreType.DMA((NBUF,)),
    ],
    compiler_params=pltpu.CompilerParams(use_tc_tiling_on_sc=False,
                                         needs_layout_passes=False),
)
def _gather_kernel(table_hbm, idx_hbm, out_hbm, idx_v, rows_v, out_pad,
                   table_sh, idx_sem, gat_sem, wb_sem):
    sid = lax.axis_index("s")
    w = sid * 2 + lax.axis_index("c")
    c0 = w * CPW

    @pl.when(sid == 0)
    def _stage_table():
        pltpu.sync_copy(table_hbm, table_sh)

    plsc.subcore_barrier()

    iota16 = lax.broadcasted_iota(jnp.int32, (16,), 0)
    row_ids = [jnp.full((16,), ii * 16, jnp.int32) + iota16 for ii in range(4)]

    def decode(c):
        return c // 512, (c % 512) // 8, (c % 8) // 2, c % 2

    def start_idx(c, buf):
        pltpu.async_copy(idx_hbm.at[pl.ds(c * CHUNK, CHUNK)],
                         idx_v.at[buf], idx_sem.at[buf])

    def wait_idx(buf):
        pltpu.make_async_copy(idx_hbm.at[pl.ds(0, CHUNK)],
                              idx_v.at[buf], idx_sem.at[buf]).wait()

    def start_gathers(buf):
        pltpu.async_copy(table_sh.at[idx_v.at[buf]], rows_v.at[buf],
                         gat_sem.at[buf])

    def wait_gathers(buf):
        pltpu.make_async_copy(table_sh.at[idx_v.at[buf]], rows_v.at[buf],
                              gat_sem.at[buf]).wait()

    def start_wb(c, buf):
        b, it, jt, half = decode(c)
        for ii in range(4):
            for ht in range(2):
                pltpu.async_copy(
                    out_pad.at[buf, pl.ds(ii * 16 + ht * 8, 8),
                               pl.ds(0, 128)],
                    out_hbm.at[b, it * 8 + half * 4 + ii, ht, jt],
                    wb_sem.at[buf])

    def wait_wb(buf):
        for ii in range(4):
            for ht in range(2):
                pltpu.make_async_copy(
                    out_pad.at[buf, pl.ds(ii * 16 + ht * 8, 8),
                               pl.ds(0, 128)],
                    out_hbm.at[0, 0, ht, 0],
                    wb_sem.at[buf]).wait()

    def compute(buf):
        @plsc.parallel_loop(0, 128, 1, unroll=4)
        def _body(jj):
            col = jnp.full((16,), jj, jnp.int32)
            for ii in range(4):
                vals = rows_v[buf, ii * 128 + jj]
                plsc.store_scatter(out_pad.at[buf], [row_ids[ii], col], vals)

    def run_chunk(c, buf, skip_wb_wait, has1, hasn):
        if has1:
            wait_idx((buf + 1) % NBUF)
            start_gathers((buf + 1) % NBUF)
        wait_gathers(buf)
        if not skip_wb_wait:
            wait_wb(buf)
        compute(buf)
        start_wb(c, buf)
        if hasn:
            start_idx(c + NBUF, buf)

    for k in range(NBUF):
        start_idx(c0 + k, k)
    wait_idx(0)
    start_gathers(0)
    for k in range(NBUF):
        run_chunk(c0 + k, k, True, True, True)

    n_steady = (CPW - 2 * NBUF) // NBUF

    def rounds(r, carry):
        g = c0 + NBUF + NBUF * r
        for k in range(NBUF):
            run_chunk(g + k, k, False, True, True)
        return carry

    lax.fori_loop(0, n_steady, rounds, 0)

    for g in range(NBUF + n_steady * NBUF, CPW):
        run_chunk(c0 + g, g % NBUF, False, g + 1 < CPW, g + NBUF < CPW)
    for k in range(NBUF):
        wait_wb(k)


def kernel(dist, embedding_table):
    idx = (dist.astype(jnp.int32)
           .reshape(8, 64, 8, 4, 128)
           .transpose(0, 1, 3, 2, 4)
           .reshape(B_TOTAL))
    out = _gather_kernel(embedding_table, idx)
    return out.transpose(0, 1, 3, 5, 2, 4).reshape(8, 512, 512, NUM_HEADS)
